# baseline (speedup 1.0000x reference)
import functools

import jax
import jax.numpy as jnp
import numpy as np
from jax import lax
from jax.experimental import pallas as pl
from jax.experimental.pallas import tpu as pltpu
from jax.experimental.pallas import tpu_sc as plsc

B, L, V, D, NB = 1024, 200, 1000000, 128, 32
N = B * L

NC, NS = 2, 16
NW = NC * NS
ROWS_PER_W = N // NW
CHUNK = 128
BUF_ROWS = 640
N_OUTER = ROWS_PER_W // BUF_ROWS
N_INNER = BUF_ROWS // CHUNK

ROWS_TC = 2048
TC_GRID = N // ROWS_TC


@functools.partial(
    pl.kernel,
    out_type=jax.ShapeDtypeStruct((N, D), jnp.float32),
    mesh=plsc.VectorSubcoreMesh(core_axis_name="c", subcore_axis_name="s"),
    scratch_types=[
        pltpu.VMEM((ROWS_PER_W,), jnp.int32),
        pltpu.VMEM((BUF_ROWS, D), jnp.float32),
        pltpu.SemaphoreType.DMA,
    ],
)
def _sc_gather(idx_hbm, table_hbm, out_hbm, idx_v, rows_v, sem):
    wid = lax.axis_index("s") * NC + lax.axis_index("c")
    base = wid * ROWS_PER_W
    pltpu.sync_copy(idx_hbm.at[pl.ds(base, ROWS_PER_W)], idx_v)

    def outer(g, carry):
        row0 = g * BUF_ROWS
        cps = []
        for j in range(N_INNER):
            idx_slice = idx_v.at[pl.ds(row0 + j * CHUNK, CHUNK)]
            dst = rows_v.at[pl.ds(j * CHUNK, CHUNK)]
            cps.append(pltpu.async_copy(table_hbm.at[idx_slice], dst, sem))
        for cp in cps:
            cp.wait()
        pltpu.sync_copy(rows_v, out_hbm.at[pl.ds(base + row0, BUF_ROWS)])
        return carry

    lax.fori_loop(0, N_OUTER, outer, 0)


def _tc_body(af_ref, tok_ref, freq_ref, w_ref, b_ref, out_ref):
    a = af_ref[...]
    ang = a * freq_ref[...]
    is_sin = lax.broadcasted_iota(jnp.int32, ang.shape, 1) < NB
    feats = jnp.where(is_sin, jnp.sin(ang), jnp.cos(ang))
    proj = jnp.dot(feats, w_ref[...], preferred_element_type=jnp.float32)
    out_ref[...] = proj + b_ref[...] + tok_ref[...]


def _tc_embed(af_col, tok, freq, af_W, af_b2):
    return pl.pallas_call(
        _tc_body,
        grid=(TC_GRID,),
        in_specs=[
            pl.BlockSpec((ROWS_TC, 1), lambda i: (i, 0)),
            pl.BlockSpec((ROWS_TC, D), lambda i: (i, 0)),
            pl.BlockSpec((1, 2 * NB), lambda i: (0, 0)),
            pl.BlockSpec((2 * NB, D), lambda i: (0, 0)),
            pl.BlockSpec((1, D), lambda i: (0, 0)),
        ],
        out_specs=pl.BlockSpec((ROWS_TC, D), lambda i: (i, 0)),
        out_shape=jax.ShapeDtypeStruct((N, D), jnp.float32),
    )(af_col, tok, freq, af_W, af_b2)


_FREQS = np.concatenate([(2.0 ** np.arange(NB)) * np.pi] * 2).astype(np.float32)


@jax.jit
def kernel(seq, af, table, af_W, af_b):
    idx = seq.reshape(N)
    tok = _sc_gather(idx, table)
    af_col = af.reshape(N, 1)
    freq = jnp.asarray(_FREQS).reshape(1, 2 * NB)
    out = _tc_embed(af_col, tok, freq, af_W, af_b.reshape(1, D))
    return out.reshape(B, L, D)


# speedup vs baseline: 1.1570x; 1.1570x over previous
import functools

import jax
import jax.numpy as jnp
import numpy as np
from jax import lax
from jax.experimental import pallas as pl
from jax.experimental.pallas import tpu as pltpu
from jax.experimental.pallas import tpu_sc as plsc

B, L, V, D, NB = 1024, 200, 1000000, 128, 32
N = B * L

NC, NS = 2, 16
NW = NC * NS
ROWS_PER_W = N // NW
CHUNK = 80
BUF_ROWS = 320
N_INNER = BUF_ROWS // CHUNK
N_GROUPS = ROWS_PER_W // BUF_ROWS

R4 = 512
TC_GRID = (N // 4) // R4


@functools.partial(
    pl.kernel,
    out_type=jax.ShapeDtypeStruct((N, D), jnp.float32),
    mesh=plsc.VectorSubcoreMesh(core_axis_name="c", subcore_axis_name="s"),
    scratch_types=[
        pltpu.VMEM((ROWS_PER_W,), jnp.int32),
        pltpu.VMEM((BUF_ROWS, D), jnp.float32),
        pltpu.VMEM((BUF_ROWS, D), jnp.float32),
        pltpu.SemaphoreType.DMA,
        pltpu.SemaphoreType.DMA,
        pltpu.SemaphoreType.DMA,
    ],
)
def _sc_gather(idx_hbm, table_hbm, out_hbm, idx_v, rows0, rows1, gsem, wsem0, wsem1):
    wid = lax.axis_index("s") * NC + lax.axis_index("c")
    base = wid * ROWS_PER_W
    pltpu.sync_copy(idx_hbm.at[pl.ds(base, ROWS_PER_W)], idx_v)
    bufs = (rows0, rows1)
    wsems = (wsem0, wsem1)

    def outer(i, carry):
        for b in range(2):
            g = 2 * i + b
            row0 = g * BUF_ROWS
            rows_b, wsem_b = bufs[b], wsems[b]

            @pl.when(g >= 2)
            def _():
                pltpu.make_async_copy(
                    rows_b, out_hbm.at[pl.ds(base, BUF_ROWS)], wsem_b
                ).wait()

            cps = []
            for j in range(N_INNER):
                idx_slice = idx_v.at[pl.ds(row0 + j * CHUNK, CHUNK)]
                dst = rows_b.at[pl.ds(j * CHUNK, CHUNK)]
                cps.append(pltpu.async_copy(table_hbm.at[idx_slice], dst, gsem))
            for cp in cps:
                cp.wait()
            pltpu.async_copy(rows_b, out_hbm.at[pl.ds(base + row0, BUF_ROWS)], wsem_b)
        return carry

    lax.fori_loop(0, N_GROUPS // 2, outer, 0)
    for b in range(2):
        pltpu.make_async_copy(
            bufs[b], out_hbm.at[pl.ds(base, BUF_ROWS)], wsems[b]
        ).wait()


def _tc_body(af4_ref, tok_ref, freq_ref, w_ref, b_ref, out_ref):
    af4 = af4_ref[...]
    lane = lax.broadcasted_iota(jnp.int32, (R4, D), 1) // NB
    afx = jnp.where(
        lane == 0, af4[:, 0:1],
        jnp.where(lane == 1, af4[:, 1:2],
                  jnp.where(lane == 2, af4[:, 2:3], af4[:, 3:4])))
    ang = afx * freq_ref[...]
    feats = jnp.concatenate([jnp.sin(ang), jnp.cos(ang)], axis=1)
    proj = jnp.dot(feats, w_ref[...], preferred_element_type=jnp.float32)
    out_ref[...] = proj + b_ref[...] + tok_ref[...]


def _tc_embed(af4, tok4, freq, w_all, b4):
    return pl.pallas_call(
        _tc_body,
        grid=(TC_GRID,),
        in_specs=[
            pl.BlockSpec((R4, 4), lambda i: (i, 0)),
            pl.BlockSpec((R4, 4 * D), lambda i: (i, 0)),
            pl.BlockSpec((1, D), lambda i: (0, 0)),
            pl.BlockSpec((2 * D, 4 * D), lambda i: (0, 0)),
            pl.BlockSpec((1, 4 * D), lambda i: (0, 0)),
        ],
        out_specs=pl.BlockSpec((R4, 4 * D), lambda i: (i, 0)),
        out_shape=jax.ShapeDtypeStruct((N // 4, 4 * D), jnp.float32),
    )(af4, tok4, freq, w_all, b4)


_FREQ128 = np.tile((2.0 ** np.arange(NB)) * np.pi, 4).astype(np.float32)


def _build_w_all(af_W):
    ws, wc = af_W[:NB], af_W[NB:]
    zero = jnp.zeros((NB, D), jnp.float32)
    def bd(w):
        rows = []
        for g in range(4):
            rows.append(jnp.concatenate(
                [w if gg == g else zero for gg in range(4)], axis=1))
        return jnp.concatenate(rows, axis=0)
    return jnp.concatenate([bd(ws), bd(wc)], axis=0)


@jax.jit
def kernel(seq, af, table, af_W, af_b):
    idx = seq.reshape(N)
    tok = _sc_gather(idx, table)
    tok4 = tok.reshape(N // 4, 4 * D)
    af4 = af.reshape(N // 4, 4)
    freq = jnp.asarray(_FREQ128).reshape(1, D)
    w_all = _build_w_all(af_W)
    b4 = jnp.tile(af_b, 4).reshape(1, 4 * D)
    out = _tc_embed(af4, tok4, freq, w_all, b4)
    return out.reshape(B, L, D)


# speedup vs baseline: 2.0858x; 1.8027x over previous
import functools

import jax
import jax.numpy as jnp
import numpy as np
from jax import lax
from jax.experimental import pallas as pl
from jax.experimental.pallas import tpu as pltpu
from jax.experimental.pallas import tpu_sc as plsc

B, L, V, D, NB = 1024, 200, 1000000, 128, 32
N = B * L

NC, NS = 2, 16
NW = NC * NS
ROWS_PER_W = N // NW
CHUNK = 80
BUF_ROWS = 320
N_INNER = BUF_ROWS // CHUNK
N_GROUPS = ROWS_PER_W // BUF_ROWS

R4 = 512
TC_GRID = (N // 4) // R4


@functools.partial(
    pl.kernel,
    out_type=jax.ShapeDtypeStruct((N, D), jnp.float32),
    mesh=plsc.VectorSubcoreMesh(core_axis_name="c", subcore_axis_name="s"),
    scratch_types=[
        pltpu.VMEM((ROWS_PER_W,), jnp.int32),
        pltpu.VMEM((BUF_ROWS, D), jnp.float32),
        pltpu.VMEM((BUF_ROWS, D), jnp.float32),
        pltpu.SemaphoreType.DMA,
        pltpu.SemaphoreType.DMA,
        pltpu.SemaphoreType.DMA,
    ],
)
def _sc_gather(idx_hbm, table_hbm, out_hbm, idx_v, rows0, rows1, gsem, wsem0, wsem1):
    wid = lax.axis_index("s") * NC + lax.axis_index("c")
    base = wid * ROWS_PER_W
    pltpu.sync_copy(idx_hbm.at[pl.ds(base, ROWS_PER_W)], idx_v)
    bufs = (rows0, rows1)
    wsems = (wsem0, wsem1)

    def outer(i, carry):
        for b in range(2):
            g = 2 * i + b
            row0 = g * BUF_ROWS
            rows_b, wsem_b = bufs[b], wsems[b]

            @pl.when(g >= 2)
            def _():
                pltpu.make_async_copy(
                    rows_b, out_hbm.at[pl.ds(base, BUF_ROWS)], wsem_b
                ).wait()

            cps = []
            for j in range(N_INNER):
                idx_slice = idx_v.at[pl.ds(row0 + j * CHUNK, CHUNK)]
                dst = rows_b.at[pl.ds(j * CHUNK, CHUNK)]
                cps.append(pltpu.async_copy(table_hbm.at[idx_slice], dst, gsem))
            for cp in cps:
                cp.wait()
            pltpu.async_copy(rows_b, out_hbm.at[pl.ds(base + row0, BUF_ROWS)], wsem_b)
        return carry

    lax.fori_loop(0, N_GROUPS // 2, outer, 0)
    for b in range(2):
        pltpu.make_async_copy(
            bufs[b], out_hbm.at[pl.ds(base, BUF_ROWS)], wsems[b]
        ).wait()


def _tc_body(af4_ref, tok_ref, freq_ref, w_ref, b_ref, out_ref):
    af4 = af4_ref[...]
    lane = lax.broadcasted_iota(jnp.int32, (R4, D), 1) // NB
    afx = jnp.where(
        lane == 0, af4[:, 0:1],
        jnp.where(lane == 1, af4[:, 1:2],
                  jnp.where(lane == 2, af4[:, 2:3], af4[:, 3:4])))
    ang = afx * freq_ref[...]
    w = w_ref[...]
    proj4 = (
        jnp.dot(jnp.sin(ang), w[:D, :], preferred_element_type=jnp.float32)
        + jnp.dot(jnp.cos(ang), w[D:, :], preferred_element_type=jnp.float32)
    )
    proj = jnp.concatenate(
        [proj4[:, g * D:(g + 1) * D] for g in range(4)], axis=0)
    out_ref[...] = proj + b_ref[...] + tok_ref[...]


def _tc_embed(af4, tok, freq, w_all, b2):
    return pl.pallas_call(
        _tc_body,
        grid=(TC_GRID,),
        in_specs=[
            pl.BlockSpec((R4, 4), lambda i: (i, 0)),
            pl.BlockSpec((4 * R4, D), lambda i: (i, 0)),
            pl.BlockSpec((1, D), lambda i: (0, 0)),
            pl.BlockSpec((2 * D, 4 * D), lambda i: (0, 0)),
            pl.BlockSpec((1, D), lambda i: (0, 0)),
        ],
        out_specs=pl.BlockSpec((4 * R4, D), lambda i: (i, 0)),
        out_shape=jax.ShapeDtypeStruct((N, D), jnp.float32),
    )(af4, tok, freq, w_all, b2)


_FREQ128 = np.tile((2.0 ** np.arange(NB)) * np.pi, 4).astype(np.float32)


def _build_w_all(af_W):
    ws, wc = af_W[:NB], af_W[NB:]
    zero = jnp.zeros((NB, D), jnp.float32)
    def bd(w):
        rows = []
        for g in range(4):
            rows.append(jnp.concatenate(
                [w if gg == g else zero for gg in range(4)], axis=1))
        return jnp.concatenate(rows, axis=0)
    return jnp.concatenate([bd(ws), bd(wc)], axis=0)


@jax.jit
def kernel(seq, af, table, af_W, af_b):
    idx = seq.reshape(N)
    tok = _sc_gather(idx, table)
    af4 = af.reshape(TC_GRID, 4, R4).transpose(0, 2, 1).reshape(N // 4, 4)
    freq = jnp.asarray(_FREQ128).reshape(1, D)
    w_all = _build_w_all(af_W)
    out = _tc_embed(af4, tok, freq, w_all, af_b.reshape(1, D))
    return out.reshape(B, L, D)


# speedup vs baseline: 2.1120x; 1.0126x over previous
import functools

import jax
import jax.numpy as jnp
import numpy as np
from jax import lax
from jax.experimental import pallas as pl
from jax.experimental.pallas import tpu as pltpu
from jax.experimental.pallas import tpu_sc as plsc

B, L, V, D, NB = 1024, 200, 1000000, 128, 32
N = B * L

C = 2
NCHK = N // C

NC, NS = 2, 16
NW = NC * NS
CHUNK = 80

R4 = 512
BLK = 4 * R4
TC_GRID = N // BLK
TC_GRID_C = TC_GRID // C


def _make_sc_gather(rows_total, buf_rows):
    rows_per_w = rows_total // NW
    n_groups = rows_per_w // buf_rows
    n_inner = buf_rows // CHUNK
    assert rows_per_w % buf_rows == 0 and n_groups % 2 == 0
    assert buf_rows % CHUNK == 0 and rows_per_w % 8 == 0

    @functools.partial(
        pl.kernel,
        out_type=jax.ShapeDtypeStruct((rows_total, D), jnp.float32),
        mesh=plsc.VectorSubcoreMesh(core_axis_name="c", subcore_axis_name="s"),
        scratch_types=[
            pltpu.VMEM((rows_per_w,), jnp.int32),
            pltpu.VMEM((buf_rows, D), jnp.float32),
            pltpu.VMEM((buf_rows, D), jnp.float32),
            pltpu.SemaphoreType.DMA,
            pltpu.SemaphoreType.DMA,
            pltpu.SemaphoreType.DMA,
        ],
    )
    def sc_gather(idx_hbm, table_hbm, out_hbm, idx_v, rows0, rows1,
                  gsem, wsem0, wsem1):
        wid = lax.axis_index("s") * NC + lax.axis_index("c")
        base = wid * rows_per_w
        pltpu.sync_copy(idx_hbm.at[pl.ds(base, rows_per_w)], idx_v)
        bufs = (rows0, rows1)
        wsems = (wsem0, wsem1)

        def outer(i, carry):
            for b in range(2):
                g = 2 * i + b
                row0 = g * buf_rows
                rows_b, wsem_b = bufs[b], wsems[b]

                @pl.when(g >= 2)
                def _():
                    pltpu.make_async_copy(
                        rows_b, out_hbm.at[pl.ds(base, buf_rows)], wsem_b
                    ).wait()

                cps = []
                for j in range(n_inner):
                    idx_slice = idx_v.at[pl.ds(row0 + j * CHUNK, CHUNK)]
                    dst = rows_b.at[pl.ds(j * CHUNK, CHUNK)]
                    cps.append(
                        pltpu.async_copy(table_hbm.at[idx_slice], dst, gsem))
                for cp in cps:
                    cp.wait()
                pltpu.async_copy(
                    rows_b, out_hbm.at[pl.ds(base + row0, buf_rows)], wsem_b)
            return carry

        lax.fori_loop(0, n_groups // 2, outer, 0)
        for b in range(2):
            pltpu.make_async_copy(
                bufs[b], out_hbm.at[pl.ds(base, buf_rows)], wsems[b]
            ).wait()

    return sc_gather


_sc_gather_chunk = _make_sc_gather(NCHK, 320)


def _afx(af4):
    lane = lax.broadcasted_iota(jnp.int32, (R4, D), 1) // NB
    return jnp.where(
        lane == 0, af4[:, 0:1],
        jnp.where(lane == 1, af4[:, 1:2],
                  jnp.where(lane == 2, af4[:, 2:3], af4[:, 3:4])))


def _tc_body_first(af4_ref, tok_ref, freq_ref, w_ref, b_ref, out_ref):
    _tc_common(af4_ref, tok_ref, freq_ref, w_ref, b_ref, out_ref)


def _tc_body_chained(buf_ref, af4_ref, tok_ref, freq_ref, w_ref, b_ref,
                     out_ref):
    del buf_ref
    _tc_common(af4_ref, tok_ref, freq_ref, w_ref, b_ref, out_ref)


def _tc_common(af4_ref, tok_ref, freq_ref, w_ref, b_ref, out_ref):
    af4 = af4_ref[...]
    ang = _afx(af4) * freq_ref[...]
    w = w_ref[...]
    proj4 = (
        jnp.dot(jnp.sin(ang), w[:D, :], preferred_element_type=jnp.float32)
        + jnp.dot(jnp.cos(ang), w[D:, :], preferred_element_type=jnp.float32)
    )
    proj = jnp.concatenate(
        [proj4[:, g * D:(g + 1) * D] for g in range(4)], axis=0)
    out_ref[...] = proj + b_ref[...] + tok_ref[...]


def _tc_embed_chunk(c, buf, af4_c, tok_c, freq, w_all, b2):
    common_specs = [
        pl.BlockSpec((R4, 4), lambda i: (i, 0)),
        pl.BlockSpec((BLK, D), lambda i: (i, 0)),
        pl.BlockSpec((1, D), lambda i: (0, 0)),
        pl.BlockSpec((2 * D, 4 * D), lambda i: (0, 0)),
        pl.BlockSpec((1, D), lambda i: (0, 0)),
    ]
    out_spec = pl.BlockSpec((BLK, D), lambda i: (i + c * TC_GRID_C, 0))
    out_shape = jax.ShapeDtypeStruct((N, D), jnp.float32)
    if buf is None:
        return pl.pallas_call(
            _tc_body_first,
            grid=(TC_GRID_C,),
            in_specs=common_specs,
            out_specs=out_spec,
            out_shape=out_shape,
        )(af4_c, tok_c, freq, w_all, b2)
    return pl.pallas_call(
        _tc_body_chained,
        grid=(TC_GRID_C,),
        in_specs=[pl.BlockSpec(memory_space=pltpu.MemorySpace.HBM)] + common_specs,
        out_specs=out_spec,
        out_shape=out_shape,
        input_output_aliases={0: 0},
    )(buf, af4_c, tok_c, freq, w_all, b2)


_FREQ128 = np.tile((2.0 ** np.arange(NB)) * np.pi, 4).astype(np.float32)


def _build_w_all(af_W):
    ws, wc = af_W[:NB], af_W[NB:]
    zero = jnp.zeros((NB, D), jnp.float32)
    def bd(w):
        rows = []
        for g in range(4):
            rows.append(jnp.concatenate(
                [w if gg == g else zero for gg in range(4)], axis=1))
        return jnp.concatenate(rows, axis=0)
    return jnp.concatenate([bd(ws), bd(wc)], axis=0)


@jax.jit
def kernel(seq, af, table, af_W, af_b):
    idx = seq.reshape(N)
    af4 = af.reshape(TC_GRID, 4, R4).transpose(0, 2, 1).reshape(N // 4, 4)
    freq = jnp.asarray(_FREQ128).reshape(1, D)
    w_all = _build_w_all(af_W)
    b2 = af_b.reshape(1, D)

    toks = [
        _sc_gather_chunk(lax.slice(idx, (c * NCHK,), ((c + 1) * NCHK,)), table)
        for c in range(C)
    ]
    buf = None
    for c in range(C):
        af4_c = lax.slice(af4, (c * NCHK // 4, 0), ((c + 1) * NCHK // 4, 4))
        buf = _tc_embed_chunk(c, buf, af4_c, toks[c], freq, w_all, b2)
    return buf.reshape(B, L, D)


# speedup vs baseline: 2.3485x; 1.1120x over previous
import functools

import jax
import jax.numpy as jnp
import numpy as np
from jax import lax
from jax.experimental import pallas as pl
from jax.experimental.pallas import tpu as pltpu
from jax.experimental.pallas import tpu_sc as plsc

B, L, V, D, NB = 1024, 200, 1000000, 128, 32
N = B * L

C = 4
NCHK = N // C

NC, NS = 2, 16
NW = NC * NS
CHUNK = 80
BUF_ROWS = 160

R4 = 512
BLK = 4 * R4
TC_GRID = N // BLK
TC_GRID_C = TC_GRID // C


def _make_sc_gather(chunk_offset, rows_total, buf_rows):
    rows_per_w = rows_total // NW
    n_groups = rows_per_w // buf_rows
    n_inner = buf_rows // CHUNK
    assert rows_per_w % buf_rows == 0 and n_groups % 2 == 0
    assert buf_rows % CHUNK == 0 and rows_per_w % 8 == 0

    @functools.partial(
        pl.kernel,
        out_type=jax.ShapeDtypeStruct((rows_total, D), jnp.float32),
        mesh=plsc.VectorSubcoreMesh(core_axis_name="c", subcore_axis_name="s"),
        scratch_types=[
            pltpu.VMEM((rows_per_w,), jnp.int32),
            pltpu.VMEM((buf_rows, D), jnp.float32),
            pltpu.VMEM((buf_rows, D), jnp.float32),
            pltpu.SemaphoreType.DMA,
            pltpu.SemaphoreType.DMA,
            pltpu.SemaphoreType.DMA,
        ],
    )
    def sc_gather(idx_hbm, table_hbm, out_hbm, idx_v, rows0, rows1,
                  gsem, wsem0, wsem1):
        wid = lax.axis_index("s") * NC + lax.axis_index("c")
        base = wid * rows_per_w
        pltpu.sync_copy(idx_hbm.at[pl.ds(chunk_offset + base, rows_per_w)],
                        idx_v)
        bufs = (rows0, rows1)
        wsems = (wsem0, wsem1)

        def outer(i, carry):
            for b in range(2):
                g = 2 * i + b
                row0 = g * buf_rows
                rows_b, wsem_b = bufs[b], wsems[b]

                @pl.when(g >= 2)
                def _():
                    pltpu.make_async_copy(
                        rows_b, out_hbm.at[pl.ds(base, buf_rows)], wsem_b
                    ).wait()

                cps = []
                for j in range(n_inner):
                    idx_slice = idx_v.at[pl.ds(row0 + j * CHUNK, CHUNK)]
                    dst = rows_b.at[pl.ds(j * CHUNK, CHUNK)]
                    cps.append(
                        pltpu.async_copy(table_hbm.at[idx_slice], dst, gsem))
                for cp in cps:
                    cp.wait()
                pltpu.async_copy(
                    rows_b, out_hbm.at[pl.ds(base + row0, buf_rows)], wsem_b)
            return carry

        lax.fori_loop(0, n_groups // 2, outer, 0)
        for b in range(2):
            pltpu.make_async_copy(
                bufs[b], out_hbm.at[pl.ds(base, buf_rows)], wsems[b]
            ).wait()

    return sc_gather


_sc_gather_chunks = [_make_sc_gather(c * NCHK, NCHK, BUF_ROWS)
                     for c in range(C)]


def _tc_common(afr_ref, tok_ref, freq_ref, w_ref, b_ref, out_ref):
    af4 = jnp.transpose(afr_ref[0], (1, 0))
    lane = lax.broadcasted_iota(jnp.int32, (R4, D), 1) // NB
    afx = jnp.where(
        lane == 0, af4[:, 0:1],
        jnp.where(lane == 1, af4[:, 1:2],
                  jnp.where(lane == 2, af4[:, 2:3], af4[:, 3:4])))
    ang = afx * freq_ref[...]
    w = w_ref[...]
    proj4 = (
        jnp.dot(jnp.sin(ang), w[:D, :], preferred_element_type=jnp.float32)
        + jnp.dot(jnp.cos(ang), w[D:, :], preferred_element_type=jnp.float32)
    )
    proj = jnp.concatenate(
        [proj4[:, g * D:(g + 1) * D] for g in range(4)], axis=0)
    out_ref[...] = proj + b_ref[...] + tok_ref[...]


def _tc_body_first(afr_ref, tok_ref, freq_ref, w_ref, b_ref, out_ref):
    _tc_common(afr_ref, tok_ref, freq_ref, w_ref, b_ref, out_ref)


def _tc_body_chained(buf_ref, afr_ref, tok_ref, freq_ref, w_ref, b_ref,
                     out_ref):
    del buf_ref
    _tc_common(afr_ref, tok_ref, freq_ref, w_ref, b_ref, out_ref)


def _tc_embed_chunk(c, buf, af_rows, tok_c, freq, w_all, b2):
    common_specs = [
        pl.BlockSpec((1, 4, 4 * D), lambda i, c=c: (i + c * TC_GRID_C, 0, 0)),
        pl.BlockSpec((BLK, D), lambda i: (i, 0)),
        pl.BlockSpec((1, D), lambda i: (0, 0)),
        pl.BlockSpec((2 * D, 4 * D), lambda i: (0, 0)),
        pl.BlockSpec((1, D), lambda i: (0, 0)),
    ]
    out_spec = pl.BlockSpec((BLK, D), lambda i, c=c: (i + c * TC_GRID_C, 0))
    out_shape = jax.ShapeDtypeStruct((N, D), jnp.float32)
    if buf is None:
        return pl.pallas_call(
            _tc_body_first,
            grid=(TC_GRID_C,),
            in_specs=common_specs,
            out_specs=out_spec,
            out_shape=out_shape,
        )(af_rows, tok_c, freq, w_all, b2)
    return pl.pallas_call(
        _tc_body_chained,
        grid=(TC_GRID_C,),
        in_specs=[pl.BlockSpec(memory_space=pltpu.MemorySpace.HBM)]
        + common_specs,
        out_specs=out_spec,
        out_shape=out_shape,
        input_output_aliases={0: 0},
    )(buf, af_rows, tok_c, freq, w_all, b2)


_FREQ128 = np.tile((2.0 ** np.arange(NB)) * np.pi, 4).astype(np.float32)


def _build_w_all(af_W):
    ws, wc = af_W[:NB], af_W[NB:]
    zero = jnp.zeros((NB, D), jnp.float32)
    def bd(w):
        rows = []
        for g in range(4):
            rows.append(jnp.concatenate(
                [w if gg == g else zero for gg in range(4)], axis=1))
        return jnp.concatenate(rows, axis=0)
    return jnp.concatenate([bd(ws), bd(wc)], axis=0)


@jax.jit
def kernel(seq, af, table, af_W, af_b):
    idx = seq.reshape(N)
    af_rows = af.reshape(TC_GRID, 4, 4 * D)
    freq = jnp.asarray(_FREQ128).reshape(1, D)
    w_all = _build_w_all(af_W)
    b2 = af_b.reshape(1, D)

    toks = [_sc_gather_chunks[c](idx, table) for c in range(C)]
    buf = None
    for c in range(C):
        buf = _tc_embed_chunk(c, buf, af_rows, toks[c], freq, w_all, b2)
    return buf.reshape(B, L, D)
